# baseline (speedup 1.0000x reference)
import jax
import jax.numpy as jnp
from jax import lax
from jax.experimental import pallas as pl
from jax.experimental.pallas import tpu as pltpu
from jax.experimental.pallas import tpu_sc as plsc

_B = 16384
_D = 32
_PACK = 4
_NC, _NS = 2, 16
_NW = _NC * _NS
_BPW = _B // _NW
_C = 128
_NCH = _BPW // _C

_EPS = 1e-3


def _gather_body(ut2, mt2, blk_hbm, sub_hbm, uout_hbm, mout_hbm,
                 blk_v, sub_v, dst_v, outc_v, sem):
    wid = lax.axis_index("s") * _NC + lax.axis_index("c")
    lanes = lax.iota(jnp.int32, 16)
    for t, tab, out_hbm in ((0, ut2, uout_hbm), (1, mt2, mout_hbm)):
        pltpu.sync_copy(blk_hbm.at[t, wid], blk_v)
        pltpu.sync_copy(sub_hbm.at[t, wid], sub_v)

        def chunk_body(j, _):
            pltpu.async_copy(tab.at[blk_v.at[j]], dst_v, sem).wait()

            def grp(g, _):
                base = g * 16
                ent = base + lanes
                sv = sub_v[j, pl.ds(base, 16)] * _D
                for l in range(_D):
                    lv = jnp.full((16,), l, jnp.int32)
                    vals = plsc.load_gather(dst_v, [ent, sv + l])
                    plsc.store_scatter(outc_v, [ent, lv], vals)
                return 0

            lax.fori_loop(0, _C // 16, grp, 0)
            pltpu.sync_copy(outc_v, out_hbm.at[wid, j])
            return 0

        lax.fori_loop(0, _NCH, chunk_body, 0)


def _bn(x, g, b):
    mu = jnp.mean(x, axis=0, keepdims=True)
    var = jnp.mean(jnp.square(x - mu), axis=0, keepdims=True)
    return g * (x - mu) * lax.rsqrt(var + _EPS) + b


def _mlp_body(u_ref, m_ref,
              W1r, b1r, g1r, be1r,
              W2r, b2r, g2r, be2r,
              W3ur, W3mr, b3r, g3r, be3r,
              W4r, b4r, g4r, be4r,
              W5r, b5r, g5r, be5r,
              Wor, bor, o_ref):
    f32 = jnp.float32
    u = jnp.maximum(jnp.dot(u_ref[:], W1r[:], preferred_element_type=f32) + b1r[:], 0.0)
    u = _bn(u, g1r[:], be1r[:])
    m = jnp.maximum(jnp.dot(m_ref[:], W2r[:], preferred_element_type=f32) + b2r[:], 0.0)
    m = _bn(m, g2r[:], be2r[:])
    x = (jnp.dot(u, W3ur[:], preferred_element_type=f32)
         + jnp.dot(m, W3mr[:], preferred_element_type=f32) + b3r[:])
    x = _bn(jnp.maximum(x, 0.0), g3r[:], be3r[:])
    x = jnp.maximum(jnp.dot(x, W4r[:], preferred_element_type=f32) + b4r[:], 0.0)
    x = _bn(x, g4r[:], be4r[:])
    x = jnp.maximum(jnp.dot(x, W5r[:], preferred_element_type=f32) + b5r[:], 0.0)
    x = _bn(x, g5r[:], be5r[:])
    o_ref[:] = jax.nn.sigmoid(jnp.dot(x, Wor[:], preferred_element_type=f32) + bor[:])


def kernel(inputs, user_table, movie_table,
           W1, b1, g1, be1,
           W2, b2, g2, be2,
           W3, b3, g3, be3,
           W4, b4, g4, be4,
           W5, b5, g5, be5,
           Wo, bo):
    ut2 = user_table.reshape(user_table.shape[0] // _PACK, _PACK * _D)
    mt2 = movie_table.reshape(movie_table.shape[0] // _PACK, _PACK * _D)

    idx2 = jnp.stack([inputs[:, 0], inputs[:, 1]]).reshape(2, _NW, _NCH, _C)
    blk = idx2 >> 2
    sub = idx2 & 3

    mesh = plsc.VectorSubcoreMesh(core_axis_name="c", subcore_axis_name="s")
    u4, m4 = pl.kernel(
        _gather_body,
        out_type=[jax.ShapeDtypeStruct((_NW, _NCH, _C, _D), jnp.float32),
                  jax.ShapeDtypeStruct((_NW, _NCH, _C, _D), jnp.float32)],
        mesh=mesh,
        scratch_types=[
            pltpu.VMEM((_NCH, _C), jnp.int32),
            pltpu.VMEM((_NCH, _C), jnp.int32),
            pltpu.VMEM((_C, _PACK * _D), jnp.float32),
            pltpu.VMEM((_C, _D), jnp.float32),
            pltpu.SemaphoreType.DMA,
        ],
        compiler_params=pltpu.CompilerParams(use_tc_tiling_on_sc=True,
                                             needs_layout_passes=False),
    )(ut2, mt2, blk, sub)
    u_emb = u4.reshape(_B, _D)
    m_emb = m4.reshape(_B, _D)

    H2 = W1.shape[1]
    out = pl.pallas_call(
        _mlp_body,
        out_shape=jax.ShapeDtypeStruct((_B, 1), jnp.float32),
    )(u_emb, m_emb,
      W1, b1.reshape(1, -1), g1.reshape(1, -1), be1.reshape(1, -1),
      W2, b2.reshape(1, -1), g2.reshape(1, -1), be2.reshape(1, -1),
      W3[:H2], W3[H2:], b3.reshape(1, -1), g3.reshape(1, -1), be3.reshape(1, -1),
      W4, b4.reshape(1, -1), g4.reshape(1, -1), be4.reshape(1, -1),
      W5, b5.reshape(1, -1), g5.reshape(1, -1), be5.reshape(1, -1),
      Wo, bo.reshape(1, -1))
    return out


# speedup vs baseline: 1.9946x; 1.9946x over previous
import jax
import jax.numpy as jnp
from jax import lax
from jax.experimental import pallas as pl
from jax.experimental.pallas import tpu as pltpu
from jax.experimental.pallas import tpu_sc as plsc

_B = 16384
_D = 32
_NC, _NS = 2, 16
_RW = 62464
_SW = 1024
_NSUB = _RW // _SW
_TAIL = 16 * _RW
_V = 1000000
_CAP = 2048
_OUTROWS = _B + 64

_EPS = 1e-3


def _gather_body(utT, mtT, uidx_hbm, midx_hbm, uout, mout,
                 idx_v, cidx_v, cpos_v, slab_v, rowbuf_v, posbuf_v, sem):
    c = lax.axis_index("c")
    s = lax.axis_index("s")
    lanes = lax.iota(jnp.int32, 16)

    def init_posbuf():
        for q in range(8):
            posbuf_v[pl.ds(q * 16, 16)] = jnp.full((16,), _B, jnp.int32)

    def flush(out_hbm):
        pltpu.async_copy(rowbuf_v, out_hbm.at[posbuf_v], sem).wait()
        init_posbuf()

    def process(tab, idx_hbm, out_hbm):
        pltpu.sync_copy(idx_hbm, idx_v)
        lo = pl.multiple_of(s * _RW, 128)
        hi = lo + _RW

        def pf(g, off):
            goff = pl.multiple_of(g * 16, 16)
            vec = idx_v[pl.ds(goff, 16)]
            inreg = (vec >= lo) & (vec < hi)
            im = inreg.astype(jnp.int32)
            csum = plsc.cumsum(im) - im
            tgt = off + csum
            plsc.store_scatter(cidx_v, [tgt], vec, mask=inreg)
            plsc.store_scatter(cpos_v, [tgt], goff + lanes, mask=inreg)
            return off + jnp.sum(im)

        cn = lax.fori_loop(0, _B // 16, pf, 0)
        ng = (cn + 15) // 16
        init_posbuf()

        def ss(si, hoff):
            c0 = pl.multiple_of(lo + si * _SW, 128)
            pltpu.sync_copy(tab.at[:, pl.ds(c0, _SW)], slab_v)

            def kg(k, h):
                koff = pl.multiple_of(k * 16, 16)
                cvec = cidx_v[pl.ds(koff, 16)]
                pvec = cpos_v[pl.ds(koff, 16)]
                local = cvec - c0
                m = (local >= 0) & (local < _SW)
                im = m.astype(jnp.int32)
                cnt = jnp.sum(im)
                lsafe = jnp.where(m, local, 0)

                def do_extract():
                    rt = h + plsc.cumsum(im) - im
                    for d in range(_D):
                        dv = jnp.full((16,), d, jnp.int32)
                        vals = plsc.load_gather(slab_v, [dv, lsafe], mask=m)
                        plsc.store_scatter(rowbuf_v, [rt, dv], vals, mask=m)
                    plsc.store_scatter(posbuf_v, [rt], pvec, mask=m)

                pl.when(cnt > 0)(do_extract)
                h2 = h + cnt
                pl.when(h2 >= 112)(lambda: flush(out_hbm))
                return jnp.where(h2 >= 112, 0, h2)

            return lax.fori_loop(0, ng, kg, hoff)

        hfin = lax.fori_loop(0, _NSUB, ss, 0)
        pl.when(hfin > 0)(lambda: flush(out_hbm))

    pl.when(c == 0)(lambda: process(utT, uidx_hbm, uout))
    pl.when(c == 1)(lambda: process(mtT, midx_hbm, mout))


def _bn(x, g, b):
    mu = jnp.mean(x, axis=0, keepdims=True)
    var = jnp.mean(jnp.square(x - mu), axis=0, keepdims=True)
    return g * (x - mu) * lax.rsqrt(var + _EPS) + b


def _mlp_body(u_ref, m_ref,
              W1r, b1r, g1r, be1r,
              W2r, b2r, g2r, be2r,
              W3ur, W3mr, b3r, g3r, be3r,
              W4r, b4r, g4r, be4r,
              W5r, b5r, g5r, be5r,
              Wor, bor, o_ref):
    f32 = jnp.float32
    u = jnp.maximum(jnp.dot(u_ref[:], W1r[:], preferred_element_type=f32) + b1r[:], 0.0)
    u = _bn(u, g1r[:], be1r[:])
    m = jnp.maximum(jnp.dot(m_ref[:], W2r[:], preferred_element_type=f32) + b2r[:], 0.0)
    m = _bn(m, g2r[:], be2r[:])
    x = (jnp.dot(u, W3ur[:], preferred_element_type=f32)
         + jnp.dot(m, W3mr[:], preferred_element_type=f32) + b3r[:])
    x = _bn(jnp.maximum(x, 0.0), g3r[:], be3r[:])
    x = jnp.maximum(jnp.dot(x, W4r[:], preferred_element_type=f32) + b4r[:], 0.0)
    x = _bn(x, g4r[:], be4r[:])
    x = jnp.maximum(jnp.dot(x, W5r[:], preferred_element_type=f32) + b5r[:], 0.0)
    x = _bn(x, g5r[:], be5r[:])
    o_ref[:] = jax.nn.sigmoid(jnp.dot(x, Wor[:], preferred_element_type=f32) + bor[:])


def _tail_patch(idx, tT, scanned):
    toff = idx - _TAIL
    intail = toff >= 0
    tailtab = tT[:, _TAIL:].T
    oh = (jnp.where(intail, toff, 0)[:, None]
          == jnp.arange(_V - _TAIL, dtype=jnp.int32)[None, :]).astype(jnp.float32)
    tail_rows = jnp.dot(oh, tailtab, preferred_element_type=jnp.float32)
    return jnp.where(intail[:, None], tail_rows, scanned)


def kernel(inputs, user_table, movie_table,
           W1, b1, g1, be1,
           W2, b2, g2, be2,
           W3, b3, g3, be3,
           W4, b4, g4, be4,
           W5, b5, g5, be5,
           Wo, bo):
    utT = user_table.T
    mtT = movie_table.T
    uidx = inputs[:, 0]
    midx = inputs[:, 1]

    mesh = plsc.VectorSubcoreMesh(core_axis_name="c", subcore_axis_name="s")
    out_u, out_m = pl.kernel(
        _gather_body,
        out_type=[jax.ShapeDtypeStruct((_OUTROWS, 128), jnp.float32),
                  jax.ShapeDtypeStruct((_OUTROWS, 128), jnp.float32)],
        mesh=mesh,
        scratch_types=[
            pltpu.VMEM((_B,), jnp.int32),
            pltpu.VMEM((_CAP,), jnp.int32),
            pltpu.VMEM((_CAP,), jnp.int32),
            pltpu.VMEM((_D, _SW), jnp.float32),
            pltpu.VMEM((128, 128), jnp.float32),
            pltpu.VMEM((128,), jnp.int32),
            pltpu.SemaphoreType.DMA,
        ],
        compiler_params=pltpu.CompilerParams(use_tc_tiling_on_sc=True,
                                             needs_layout_passes=False),
    )(utT, mtT, uidx, midx)

    u_emb = _tail_patch(uidx, utT, out_u[:_B, :_D])
    m_emb = _tail_patch(midx, mtT, out_m[:_B, :_D])

    H2 = W1.shape[1]
    out = pl.pallas_call(
        _mlp_body,
        out_shape=jax.ShapeDtypeStruct((_B, 1), jnp.float32),
    )(u_emb, m_emb,
      W1, b1.reshape(1, -1), g1.reshape(1, -1), be1.reshape(1, -1),
      W2, b2.reshape(1, -1), g2.reshape(1, -1), be2.reshape(1, -1),
      W3[:H2], W3[H2:], b3.reshape(1, -1), g3.reshape(1, -1), be3.reshape(1, -1),
      W4, b4.reshape(1, -1), g4.reshape(1, -1), be4.reshape(1, -1),
      W5, b5.reshape(1, -1), g5.reshape(1, -1), be5.reshape(1, -1),
      Wo, bo.reshape(1, -1))
    return out


# speedup vs baseline: 2.1992x; 1.1026x over previous
import jax
import jax.numpy as jnp
from jax import lax
from jax.experimental import pallas as pl
from jax.experimental.pallas import tpu as pltpu
from jax.experimental.pallas import tpu_sc as plsc

_B = 16384
_D = 32
_NC, _NS = 2, 16
_RW = 62464
_SW = 1024
_NSUB = _RW // _SW
_TAIL = 16 * _RW
_V = 1000000
_CAP = 2048
_OUTROWS = _B + 64

_EPS = 1e-3


def _gather_body(utT, mtT, uidx_hbm, midx_hbm, uout, mout,
                 idx_v, cidx_v, cpos_v, slab0_v, slab1_v, rowbuf_v, posbuf_v,
                 sem, fsem):
    c = lax.axis_index("c")
    s = lax.axis_index("s")
    lanes = lax.iota(jnp.int32, 16)

    def init_posbuf():
        for q in range(8):
            posbuf_v[pl.ds(q * 16, 16)] = jnp.full((16,), _B, jnp.int32)

    def flush(out_hbm):
        pltpu.async_copy(rowbuf_v, out_hbm.at[posbuf_v], fsem).wait()
        init_posbuf()

    def process(tab, idx_hbm, out_hbm):
        lo = pl.multiple_of(s * _RW, 128)
        hi = lo + _RW

        def slice_at(si):
            return tab.at[:, pl.ds(pl.multiple_of(lo + si * _SW, 128), _SW)]

        pltpu.make_async_copy(slice_at(0), slab0_v, sem).start()
        pltpu.sync_copy(idx_hbm, idx_v)

        def pf(g, off):
            goff = pl.multiple_of(g * 16, 16)
            vec = idx_v[pl.ds(goff, 16)]
            inreg = (vec >= lo) & (vec < hi)
            im = inreg.astype(jnp.int32)
            csum = plsc.cumsum(im) - im
            tgt = off + csum
            plsc.store_scatter(cidx_v, [tgt], vec, mask=inreg)
            plsc.store_scatter(cpos_v, [tgt], goff + lanes, mask=inreg)
            return off + jnp.sum(im)

        cn = lax.fori_loop(0, _B // 16, pf, 0)
        ng = (cn + 15) // 16
        init_posbuf()

        def extract(si, buf, h0):
            c0 = lo + si * _SW

            def kg(k, h):
                koff = pl.multiple_of(k * 16, 16)
                cvec = cidx_v[pl.ds(koff, 16)]
                pvec = cpos_v[pl.ds(koff, 16)]
                local = cvec - c0
                m = (local >= 0) & (local < _SW)
                im = m.astype(jnp.int32)
                cnt = jnp.sum(im)
                lsafe = jnp.where(m, local, 0)

                def do_extract():
                    rt = h + plsc.cumsum(im) - im
                    for d in range(_D):
                        dv = jnp.full((16,), d, jnp.int32)
                        vals = plsc.load_gather(buf, [dv, lsafe], mask=m)
                        plsc.store_scatter(rowbuf_v, [rt, dv], vals, mask=m)
                    plsc.store_scatter(posbuf_v, [rt], pvec, mask=m)

                pl.when(cnt > 0)(do_extract)
                h2 = h + cnt
                pl.when(h2 >= 112)(lambda: flush(out_hbm))
                return jnp.where(h2 >= 112, 0, h2)

            return lax.fori_loop(0, ng, kg, h0)

        def ring(g, h):
            s0 = g * 2
            pltpu.make_async_copy(slice_at(s0), slab0_v, sem).wait()
            pltpu.make_async_copy(slice_at(s0 + 1), slab1_v, sem).start()
            h = extract(s0, slab0_v, h)
            pltpu.make_async_copy(slice_at(s0 + 1), slab1_v, sem).wait()
            pltpu.make_async_copy(slice_at(s0 + 2), slab0_v, sem).start()
            return extract(s0 + 1, slab1_v, h)

        h = lax.fori_loop(0, (_NSUB - 1) // 2, ring, 0)
        last = _NSUB - 1
        pltpu.make_async_copy(slice_at(last), slab0_v, sem).wait()
        h = extract(last, slab0_v, h)
        pl.when(h > 0)(lambda: flush(out_hbm))

    pl.when(c == 0)(lambda: process(utT, uidx_hbm, uout))
    pl.when(c == 1)(lambda: process(mtT, midx_hbm, mout))


def _bn(x, g, b):
    mu = jnp.mean(x, axis=0, keepdims=True)
    var = jnp.mean(jnp.square(x - mu), axis=0, keepdims=True)
    return g * (x - mu) * lax.rsqrt(var + _EPS) + b


def _mlp_body(u_ref, m_ref,
              W1r, b1r, g1r, be1r,
              W2r, b2r, g2r, be2r,
              W3ur, W3mr, b3r, g3r, be3r,
              W4r, b4r, g4r, be4r,
              W5r, b5r, g5r, be5r,
              Wor, bor, o_ref):
    f32 = jnp.float32
    u = jnp.maximum(jnp.dot(u_ref[:], W1r[:], preferred_element_type=f32) + b1r[:], 0.0)
    u = _bn(u, g1r[:], be1r[:])
    m = jnp.maximum(jnp.dot(m_ref[:], W2r[:], preferred_element_type=f32) + b2r[:], 0.0)
    m = _bn(m, g2r[:], be2r[:])
    x = (jnp.dot(u, W3ur[:], preferred_element_type=f32)
         + jnp.dot(m, W3mr[:], preferred_element_type=f32) + b3r[:])
    x = _bn(jnp.maximum(x, 0.0), g3r[:], be3r[:])
    x = jnp.maximum(jnp.dot(x, W4r[:], preferred_element_type=f32) + b4r[:], 0.0)
    x = _bn(x, g4r[:], be4r[:])
    x = jnp.maximum(jnp.dot(x, W5r[:], preferred_element_type=f32) + b5r[:], 0.0)
    x = _bn(x, g5r[:], be5r[:])
    o_ref[:] = jax.nn.sigmoid(jnp.dot(x, Wor[:], preferred_element_type=f32) + bor[:])


def _tail_patch(idx, tT, scanned):
    toff = idx - _TAIL
    intail = toff >= 0
    tailtab = tT[:, _TAIL:].T
    oh = (jnp.where(intail, toff, 0)[:, None]
          == jnp.arange(_V - _TAIL, dtype=jnp.int32)[None, :]).astype(jnp.float32)
    tail_rows = jnp.dot(oh, tailtab, preferred_element_type=jnp.float32)
    return jnp.where(intail[:, None], tail_rows, scanned)


def kernel(inputs, user_table, movie_table,
           W1, b1, g1, be1,
           W2, b2, g2, be2,
           W3, b3, g3, be3,
           W4, b4, g4, be4,
           W5, b5, g5, be5,
           Wo, bo):
    utT = user_table.T
    mtT = movie_table.T
    uidx = inputs[:, 0]
    midx = inputs[:, 1]

    mesh = plsc.VectorSubcoreMesh(core_axis_name="c", subcore_axis_name="s")
    out_u, out_m = pl.kernel(
        _gather_body,
        out_type=[jax.ShapeDtypeStruct((_OUTROWS, 128), jnp.float32),
                  jax.ShapeDtypeStruct((_OUTROWS, 128), jnp.float32)],
        mesh=mesh,
        scratch_types=[
            pltpu.VMEM((_B,), jnp.int32),
            pltpu.VMEM((_CAP,), jnp.int32),
            pltpu.VMEM((_CAP,), jnp.int32),
            pltpu.VMEM((_D, _SW), jnp.float32),
            pltpu.VMEM((_D, _SW), jnp.float32),
            pltpu.VMEM((128, 128), jnp.float32),
            pltpu.VMEM((128,), jnp.int32),
            pltpu.SemaphoreType.DMA,
            pltpu.SemaphoreType.DMA,
        ],
        compiler_params=pltpu.CompilerParams(use_tc_tiling_on_sc=True,
                                             needs_layout_passes=False),
    )(utT, mtT, uidx, midx)

    u_emb = _tail_patch(uidx, utT, out_u[:_B, :_D])
    m_emb = _tail_patch(midx, mtT, out_m[:_B, :_D])

    H2 = W1.shape[1]
    out = pl.pallas_call(
        _mlp_body,
        out_shape=jax.ShapeDtypeStruct((_B, 1), jnp.float32),
    )(u_emb, m_emb,
      W1, b1.reshape(1, -1), g1.reshape(1, -1), be1.reshape(1, -1),
      W2, b2.reshape(1, -1), g2.reshape(1, -1), be2.reshape(1, -1),
      W3[:H2], W3[H2:], b3.reshape(1, -1), g3.reshape(1, -1), be3.reshape(1, -1),
      W4, b4.reshape(1, -1), g4.reshape(1, -1), be4.reshape(1, -1),
      W5, b5.reshape(1, -1), g5.reshape(1, -1), be5.reshape(1, -1),
      Wo, bo.reshape(1, -1))
    return out
